# baseline (speedup 1.0000x reference)
import functools

import jax
import jax.numpy as jnp
from jax import lax
from jax.experimental import pallas as pl
from jax.experimental.pallas import tpu as pltpu
from jax.experimental.pallas import tpu_sc as plsc

B, T, I, H, V, D = 8, 256, 1024, 8, 10000, 512

CB = 8
NCH = B // CB
TT = 128
VT = 2048
NV = (V + VT - 1) // VT
NSUB = 16
VP = 10240
IR = I // NSUB
VR = VP // NSUB
ZR = 128


def _mean_kernel(ah_ref, attnt_ref):
    attn = jnp.mean(ah_ref[0], axis=0)
    attnt_ref[0] = jnp.swapaxes(attn, 0, 1)


def _run_mean(attn_heads):
    return pl.pallas_call(
        _mean_kernel,
        grid=(B, T // TT),
        in_specs=[pl.BlockSpec((1, H, TT, I), lambda b, t: (b, 0, t, 0))],
        out_specs=pl.BlockSpec((1, I, TT), lambda b, t: (b, 0, t)),
        out_shape=jax.ShapeDtypeStruct((B, I, T), jnp.float32),
    )(attn_heads)


def _stats_kernel(attnt_ref, enc_ref, dec_ref, tar_ref, tok_ref, w_ref,
                  b_ref, pgen_ref, m_ref, z_ref, loss_ref):
    bi = pl.program_id(0)
    tj = pl.program_id(1)
    at = attnt_ref[0]
    dn_enc = (((0,), (0,)), ((), ()))
    ctx = lax.dot_general(at, enc_ref[0], dn_enc,
                          preferred_element_type=jnp.float32)
    cat = jnp.concatenate([ctx, dec_ref[0], tar_ref[0]], axis=1)
    logits = jnp.dot(cat, w_ref[...], preferred_element_type=jnp.float32)
    pg = jax.nn.sigmoid(logits + b_ref[0, 0])
    pgen_ref[...] = pg.reshape(1, 1, TT)

    tok = tok_ref[0]
    eq = (tok.reshape(I, 1) == tok.reshape(1, I)).astype(jnp.float32)
    g = lax.dot_general(at, eq, dn_enc,
                        preferred_element_type=jnp.float32)
    cnt = jnp.sum(eq, axis=0, keepdims=True)
    recip = 1.0 / cnt
    uniq = jnp.sum(recip)
    m = jnp.max(g, axis=1, keepdims=True)
    zt = jnp.sum(jnp.exp(g - m) * recip, axis=1, keepdims=True)
    z = zt + (V - uniq) * jnp.exp(-m)
    m_ref[...] = m.reshape(1, 1, TT)
    z_ref[...] = z.reshape(1, 1, TT)

    partial = jnp.sum(10.0 * jax.nn.relu(jnp.abs(pg - 0.5) - 0.45))

    @pl.when(jnp.logical_and(bi == 0, tj == 0))
    def _():
        loss_ref[...] = jnp.zeros((1, 1), jnp.float32)

    loss_ref[...] += partial.reshape(1, 1) / (B * T)


def _run_stats(attnt, enc, dec, tar, tok3, w, bvec):
    return pl.pallas_call(
        _stats_kernel,
        grid=(B, T // TT),
        in_specs=[
            pl.BlockSpec((1, I, TT), lambda b, t: (b, 0, t)),
            pl.BlockSpec((1, I, D), lambda b, t: (b, 0, 0)),
            pl.BlockSpec((1, TT, D), lambda b, t: (b, t, 0)),
            pl.BlockSpec((1, TT, D), lambda b, t: (b, t, 0)),
            pl.BlockSpec((1, 1, I), lambda b, t: (b, 0, 0)),
            pl.BlockSpec((3 * D, 1), lambda b, t: (0, 0)),
            pl.BlockSpec((1, 1), lambda b, t: (0, 0)),
        ],
        out_specs=[
            pl.BlockSpec((1, 1, TT), lambda b, t: (b, 0, t)),
            pl.BlockSpec((1, 1, TT), lambda b, t: (b, 0, t)),
            pl.BlockSpec((1, 1, TT), lambda b, t: (b, 0, t)),
            pl.BlockSpec((1, 1), lambda b, t: (0, 0)),
        ],
        out_shape=[
            jax.ShapeDtypeStruct((B, 1, T), jnp.float32),
            jax.ShapeDtypeStruct((B, 1, T), jnp.float32),
            jax.ShapeDtypeStruct((B, 1, T), jnp.float32),
            jax.ShapeDtypeStruct((1, 1), jnp.float32),
        ],
    )(attnt, enc, dec, tar, tok3, w, bvec.reshape(1, 1))


def _sc_scatter_body(boff, attnt_hbm, tok_hbm, s_hbm, table, abuf, zbuf,
                     tbuf):
    c = lax.axis_index("c")
    sid = lax.axis_index("s")

    def _zero_row(r, carry):
        for j in range(TT // 16):
            zbuf[r, pl.ds(j * 16, 16)] = jnp.zeros((16,), jnp.float32)
        return carry

    lax.fori_loop(0, ZR, _zero_row, 0)
    for k in range(VR // ZR):
        pltpu.sync_copy(zbuf, table.at[pl.ds(sid * VR + k * ZR, ZR)])
    plsc.subcore_barrier()

    for b in range(CB):
        pltpu.sync_copy(tok_hbm.at[boff + b, pl.ds(sid * IR, IR)], tbuf)
        pltpu.sync_copy(
            attnt_hbm.at[b, pl.ds(sid * IR, IR), pl.ds(c * TT, TT)], abuf)
        pltpu.sync_copy(abuf, table.at[tbuf], add=True)
        plsc.subcore_barrier()
        pltpu.sync_copy(table.at[pl.ds(sid * VR, VR)],
                        s_hbm.at[b, c, pl.ds(sid * VR, VR)])
        if b != CB - 1:
            plsc.subcore_barrier()
            pltpu.sync_copy(zbuf.at[pl.ds(0, IR)], table.at[tbuf])
            plsc.subcore_barrier()


def _run_scatter(k, attnt, tok):
    mesh = plsc.VectorSubcoreMesh(core_axis_name="c", subcore_axis_name="s")
    f = pl.kernel(
        functools.partial(_sc_scatter_body, k * CB),
        out_type=jax.ShapeDtypeStruct((CB, 2, VP, TT), jnp.float32),
        mesh=mesh,
        cost_estimate=pl.CostEstimate(
            flops=0, bytes_accessed=4 * CB * (2 * VP * TT + I * T),
            transcendentals=0),
        scratch_types=[
            pltpu.VMEM_SHARED((VP, TT), jnp.float32),
            pltpu.VMEM((IR, TT), jnp.float32),
            pltpu.VMEM((ZR, TT), jnp.float32),
            pltpu.VMEM((IR,), jnp.int32),
        ],
    )
    return f(attnt, tok)


def _mix_body(s_ref, gen_ref, pg_ref, m_ref, z_ref, ptr_ref, fin_ref):
    m = m_ref[0, 0].reshape(T, 1)
    zinv = 1.0 / z_ref[0, 0].reshape(T, 1)
    pg = pg_ref[0, 0].reshape(T, 1)
    st = jnp.concatenate(
        [jnp.swapaxes(s_ref[0, 0], 0, 1),
         jnp.swapaxes(s_ref[0, 1], 0, 1)], axis=0)
    ptr = jnp.exp(st - m) * zinv
    ptr_ref[0] = ptr
    fin_ref[0] = pg * gen_ref[0] + (1.0 - pg) * ptr


def _mix_kernel_first(s_ref, gen_ref, pg_ref, m_ref, z_ref,
                      ptr_ref, fin_ref):
    _mix_body(s_ref, gen_ref, pg_ref, m_ref, z_ref, ptr_ref, fin_ref)


def _mix_kernel_next(s_ref, gen_ref, pg_ref, m_ref, z_ref, ptr_in, fin_in,
                     ptr_ref, fin_ref):
    del ptr_in, fin_in
    _mix_body(s_ref, gen_ref, pg_ref, m_ref, z_ref, ptr_ref, fin_ref)


def _run_mix(k, s, gen, pg, m, z, ptr_prev, fin_prev):
    off = k * CB
    in_specs = [
        pl.BlockSpec((1, 2, VT, TT), lambda b, v: (b, 0, v, 0)),
        pl.BlockSpec((1, T, VT), lambda b, v: (b + off, 0, v)),
        pl.BlockSpec((1, 1, T), lambda b, v: (b, 0, 0)),
        pl.BlockSpec((1, 1, T), lambda b, v: (b, 0, 0)),
        pl.BlockSpec((1, 1, T), lambda b, v: (b, 0, 0)),
    ]
    args = [s, gen, pg, m, z]
    if ptr_prev is None:
        body = _mix_kernel_first
        aliases = {}
    else:
        body = _mix_kernel_next
        in_specs += [pl.BlockSpec(memory_space=pl.ANY),
                     pl.BlockSpec(memory_space=pl.ANY)]
        args += [ptr_prev, fin_prev]
        aliases = {5: 0, 6: 1}
    return pl.pallas_call(
        body,
        grid=(CB, NV),
        in_specs=in_specs,
        out_specs=[
            pl.BlockSpec((1, T, VT), lambda b, v: (b + off, 0, v)),
            pl.BlockSpec((1, T, VT), lambda b, v: (b + off, 0, v)),
        ],
        out_shape=[
            jax.ShapeDtypeStruct((B, T, V), jnp.float32),
            jax.ShapeDtypeStruct((B, T, V), jnp.float32),
        ],
        input_output_aliases=aliases,
    )(*args)


def kernel(inp_tokens, tar_embedded, generator_output, enc_output, dec_state,
           attn_heads, W_pgen, b_pgen):
    tok3 = inp_tokens.reshape(B, 1, I)
    attnt = _run_mean(attn_heads)
    s = _run_scatter(0, attnt, inp_tokens)
    pg, m, z, loss = _run_stats(attnt, enc_output, dec_state, tar_embedded,
                                tok3, W_pgen, b_pgen)
    ptr, fin = _run_mix(0, s, generator_output, pg, m, z, None, None)
    return fin, ptr, pg.reshape(B, T), loss.reshape(())


# speedup vs baseline: 1.0426x; 1.0426x over previous
import functools

import jax
import jax.numpy as jnp
from jax import lax
from jax.experimental import pallas as pl
from jax.experimental.pallas import tpu as pltpu
from jax.experimental.pallas import tpu_sc as plsc

B, T, I, H, V, D = 8, 256, 1024, 8, 10000, 512

TT = 128
VT = 2560
NSUB = 16
VP = 10240
NV = VP // VT
IR = I // NSUB
VR = VP // NSUB
ZR = 64


def _head_kernel(ah_ref, enc_ref, dec_ref, tar_ref, tok_ref, w_ref, b_ref,
                 attnt_ref, pgen_ref, m_ref, z_ref, loss_ref):
    bi = pl.program_id(0)
    tj = pl.program_id(1)
    attn = jnp.mean(ah_ref[0], axis=0)
    attnt_ref[0] = jnp.swapaxes(attn, 0, 1)

    ctx = jnp.dot(attn, enc_ref[0], preferred_element_type=jnp.float32)
    cat = jnp.concatenate([ctx, dec_ref[0], tar_ref[0]], axis=1)
    logits = jnp.dot(cat, w_ref[...], preferred_element_type=jnp.float32)
    pg = jax.nn.sigmoid(logits + b_ref[0, 0])
    pgen_ref[...] = pg.reshape(1, 1, TT)

    tok = tok_ref[0]
    eq = (tok.reshape(I, 1) == tok.reshape(1, I)).astype(jnp.float32)
    g = jnp.dot(attn, eq, preferred_element_type=jnp.float32)
    cnt = jnp.sum(eq, axis=0, keepdims=True)
    recip = 1.0 / cnt
    uniq = jnp.sum(recip)
    m = jnp.max(g, axis=1, keepdims=True)
    zt = jnp.sum(jnp.exp(g - m) * recip, axis=1, keepdims=True)
    z = zt + (V - uniq) * jnp.exp(-m)
    m_ref[...] = m.reshape(1, 1, TT)
    z_ref[...] = z.reshape(1, 1, TT)

    partial = jnp.sum(10.0 * jax.nn.relu(jnp.abs(pg - 0.5) - 0.45))

    @pl.when(jnp.logical_and(bi == 0, tj == 0))
    def _():
        loss_ref[...] = jnp.zeros((1, 1), jnp.float32)

    loss_ref[...] += partial.reshape(1, 1) / (B * T)


def _run_head(attn_heads, enc, dec, tar, tok3, w, bvec):
    return pl.pallas_call(
        _head_kernel,
        grid=(B, T // TT),
        in_specs=[
            pl.BlockSpec((1, H, TT, I), lambda b, t: (b, 0, t, 0)),
            pl.BlockSpec((1, I, D), lambda b, t: (b, 0, 0)),
            pl.BlockSpec((1, TT, D), lambda b, t: (b, t, 0)),
            pl.BlockSpec((1, TT, D), lambda b, t: (b, t, 0)),
            pl.BlockSpec((1, 1, I), lambda b, t: (b, 0, 0)),
            pl.BlockSpec((3 * D, 1), lambda b, t: (0, 0)),
            pl.BlockSpec((1, 1), lambda b, t: (0, 0)),
        ],
        out_specs=[
            pl.BlockSpec((1, I, TT), lambda b, t: (b, 0, t)),
            pl.BlockSpec((1, 1, TT), lambda b, t: (b, 0, t)),
            pl.BlockSpec((1, 1, TT), lambda b, t: (b, 0, t)),
            pl.BlockSpec((1, 1, TT), lambda b, t: (b, 0, t)),
            pl.BlockSpec((1, 1), lambda b, t: (0, 0)),
        ],
        out_shape=[
            jax.ShapeDtypeStruct((B, I, T), jnp.float32),
            jax.ShapeDtypeStruct((B, 1, T), jnp.float32),
            jax.ShapeDtypeStruct((B, 1, T), jnp.float32),
            jax.ShapeDtypeStruct((B, 1, T), jnp.float32),
            jax.ShapeDtypeStruct((1, 1), jnp.float32),
        ],
    )(attn_heads, enc, dec, tar, tok3, w, bvec.reshape(1, 1))


def _sc_scatter_body(attnt_hbm, tok_hbm, s_hbm, table, abuf, nbuf, zbuf,
                     tbuf, seme, semo, semt):
    c = lax.axis_index("c")
    sid = lax.axis_index("s")

    tok_copies = [
        pltpu.make_async_copy(tok_hbm.at[b, pl.ds(sid * IR, IR)],
                              tbuf.at[b], semt)
        for b in range(B)
    ]
    for cp in tok_copies:
        cp.start()
    at_copies = [
        pltpu.make_async_copy(
            attnt_hbm.at[b, pl.ds(sid * IR, IR), pl.ds(c * TT, TT)],
            abuf.at[b % 2], seme if b % 2 == 0 else semo)
        for b in range(B)
    ]
    at_copies[0].start()

    def _zero_row(r, carry):
        for j in range(TT // 16):
            zbuf[r, pl.ds(j * 16, 16)] = jnp.zeros((16,), jnp.float32)
        return carry

    lax.fori_loop(0, ZR, _zero_row, 0)
    for k in range(VR // ZR):
        pltpu.sync_copy(zbuf, table.at[pl.ds(sid * VR + k * ZR, ZR)])
    for cp in tok_copies:
        cp.wait()
    plsc.subcore_barrier()

    def _negate_row(r, carry):
        b2 = carry
        for j in range(TT // 16):
            nbuf[r, pl.ds(j * 16, 16)] = -abuf[b2, r, pl.ds(j * 16, 16)]
        return carry

    for b in range(B):
        if b + 1 < B:
            at_copies[b + 1].start()
        at_copies[b].wait()
        pltpu.sync_copy(abuf.at[b % 2], table.at[tbuf.at[b]], add=True)
        if b != B - 1:
            if b > 0:
                pltpu.sync_copy(nbuf, table.at[tbuf.at[b - 1]], add=True)
            lax.fori_loop(0, IR, _negate_row, b % 2)
        plsc.subcore_barrier()
        pltpu.sync_copy(table.at[pl.ds(sid * VR, VR)],
                        s_hbm.at[b, c, pl.ds(sid * VR, VR)])
        plsc.subcore_barrier()


def _run_scatter(attnt, tok):
    mesh = plsc.VectorSubcoreMesh(core_axis_name="c", subcore_axis_name="s")
    f = pl.kernel(
        _sc_scatter_body,
        out_type=jax.ShapeDtypeStruct((B, 2, VP, TT), jnp.float32),
        mesh=mesh,
        scratch_types=[
            pltpu.VMEM_SHARED((VP, TT), jnp.float32),
            pltpu.VMEM((2, IR, TT), jnp.float32),
            pltpu.VMEM((IR, TT), jnp.float32),
            pltpu.VMEM((ZR, TT), jnp.float32),
            pltpu.VMEM((B, IR), jnp.int32),
            pltpu.SemaphoreType.DMA,
            pltpu.SemaphoreType.DMA,
            pltpu.SemaphoreType.DMA,
        ],
        cost_estimate=pl.CostEstimate(
            flops=0, bytes_accessed=4 * B * (2 * VP * TT + I * T),
            transcendentals=0),
    )
    return f(attnt, tok)


def _mix_kernel(s_ref, gen_ref, pg_ref, m_ref, z_ref, ptr_ref, fin_ref):
    m = m_ref[0, 0].reshape(T, 1)
    zinv = 1.0 / z_ref[0, 0].reshape(T, 1)
    pg = pg_ref[0, 0].reshape(T, 1)
    st = jnp.concatenate(
        [jnp.swapaxes(s_ref[0, 0], 0, 1),
         jnp.swapaxes(s_ref[0, 1], 0, 1)], axis=0)
    ptr = jnp.exp(st - m) * zinv
    ptr_ref[0] = ptr
    fin_ref[0] = pg * gen_ref[0] + (1.0 - pg) * ptr


def _run_mix(s, gen, pg, m, z):
    return pl.pallas_call(
        _mix_kernel,
        grid=(B, NV),
        in_specs=[
            pl.BlockSpec((1, 2, VT, TT), lambda b, v: (b, 0, v, 0)),
            pl.BlockSpec((1, T, VT), lambda b, v: (b, 0, v)),
            pl.BlockSpec((1, 1, T), lambda b, v: (b, 0, 0)),
            pl.BlockSpec((1, 1, T), lambda b, v: (b, 0, 0)),
            pl.BlockSpec((1, 1, T), lambda b, v: (b, 0, 0)),
        ],
        out_specs=[
            pl.BlockSpec((1, T, VT), lambda b, v: (b, 0, v)),
            pl.BlockSpec((1, T, VT), lambda b, v: (b, 0, v)),
        ],
        out_shape=[
            jax.ShapeDtypeStruct((B, T, V), jnp.float32),
            jax.ShapeDtypeStruct((B, T, V), jnp.float32),
        ],
    )(s, gen, pg, m, z)


def kernel(inp_tokens, tar_embedded, generator_output, enc_output, dec_state,
           attn_heads, W_pgen, b_pgen):
    tok3 = inp_tokens.reshape(B, 1, I)
    attnt, pg, m, z, loss = _run_head(
        attn_heads, enc_output, dec_state, tar_embedded, tok3,
        W_pgen, b_pgen)
    s = _run_scatter(attnt, inp_tokens)
    ptr, fin = _run_mix(s, generator_output, pg, m, z)
    return fin, ptr, pg.reshape(B, T), loss.reshape(())
